# Initial kernel scaffold; baseline (speedup 1.0000x reference)
import jax
import jax.numpy as jnp
from jax.experimental import pallas as pl


def kernel(x, nn_feature, edge_attr, params, edge_index, batch):
    raise NotImplementedError("write your pallas kernel here")


# speedup vs baseline: 3.5146x; 3.5146x over previous
import functools

import jax
import jax.numpy as jnp
from jax import lax
from jax.experimental import pallas as pl
from jax.experimental.pallas import tpu as pltpu
from jax.experimental.pallas import tpu_sc as plsc

_N = 10000
_E = 320000
_K = 10
_NK = _N * _K
_NKPAD = 102400
_EPAD = 323584
_EPS = 1e-5
_F32 = jnp.float32


def _lrelu(x):
    return jnp.where(x >= 0, x, 0.01 * x)


def _dot(a, b):
    return jnp.dot(a.astype(jnp.bfloat16), b.astype(jnp.bfloat16),
                   preferred_element_type=_F32)


def _bn_host(v, gamma=None, beta=None):
    mean = jnp.mean(v, axis=0)
    var = jnp.mean((v - mean) ** 2, axis=0)
    y = (v - mean) / jnp.sqrt(var + _EPS)
    if gamma is not None:
        y = y * gamma + beta
    return y


def _bn_stats(v):
    mean = jnp.mean(v, axis=0)
    var = jnp.mean((v - mean) ** 2, axis=0)
    return mean, var


def _prelude_body(x_ref, nn_ref, new1, neb1, new2, neb2, nfw1, nfb1, nfw2,
                  nfb2, fuw, fub, box_ref, fus_ref):
    b = _lrelu(_dot(x_ref[...], new1[...]) + neb1[...])
    b = _lrelu(_dot(b, new2[...]) + neb2[...])
    box_ref[...] = b
    f = _lrelu(_dot(nn_ref[...], nfw1[...]) + nfb1[...])
    f = _lrelu(_dot(f, nfw2[...]) + nfb2[...])
    cat = jnp.concatenate([b, f], axis=1)
    fus_ref[...] = _lrelu(_dot(cat, fuw[...]) + fub[...])


def _prelude(xn, nnn, p):
    r1 = lambda v: v.reshape(1, -1)
    return pl.pallas_call(
        _prelude_body,
        out_shape=[jax.ShapeDtypeStruct((_N, 128), _F32),
                   jax.ShapeDtypeStruct((_N, 128), _F32)],
    )(xn, nnn, p['ne_w1'], r1(p['ne_b1']), p['ne_w2'], r1(p['ne_b2']),
      p['nf_w1'], r1(p['nf_b1']), p['nf_w2'], r1(p['nf_b2']),
      p['fu_w'], r1(p['fu_b']))


def _knn(h, d_feat, bb=200):
    sq = jnp.sum(h * h, axis=1)
    sq_row = sq[None, :]
    sq_col = sq[:, None]

    def body(hb_ref, h_ref, sqr_ref, sqc_ref, out_ref):
        hb = hb_ref[...]
        h_full = h_ref[...]
        dot = lax.dot_general(hb.astype(jnp.bfloat16),
                              h_full.astype(jnp.bfloat16),
                              (((1,), (1,)), ((), ())),
                              preferred_element_type=_F32)
        neg = 2.0 * dot - (sqc_ref[...] + sqr_ref[...])
        iota = lax.broadcasted_iota(jnp.int32, (bb, _N), 1)
        lane16 = lax.broadcasted_iota(jnp.int32, (bb, 16), 1)
        out = jnp.zeros((bb, 16), jnp.int32)
        for t in range(_K):
            m = jnp.max(neg, axis=1, keepdims=True)
            sel = jnp.min(jnp.where(neg == m, iota, _N), axis=1,
                          keepdims=True)
            out = jnp.where(lane16 == t, jnp.broadcast_to(sel, (bb, 16)), out)
            neg = jnp.where(iota == sel, -jnp.inf, neg)
        out_ref[...] = out

    return pl.pallas_call(
        body,
        grid=(_N // bb,),
        in_specs=[pl.BlockSpec((bb, d_feat), lambda i: (i, 0)),
                  pl.BlockSpec((_N, d_feat), lambda i: (0, 0)),
                  pl.BlockSpec((1, _N), lambda i: (0, 0)),
                  pl.BlockSpec((bb, 1), lambda i: (i, 0))],
        out_specs=pl.BlockSpec((bb, 16), lambda i: (i, 0)),
        out_shape=jax.ShapeDtypeStruct((_N, 16), jnp.int32),
    )(h, h, sq_row, sq_col)


def _sc_gather(table, idx_pad):
    b_tot, d_feat = idx_pad.shape[0], table.shape[1]
    nw, ch = 32, 128
    per_w = b_tot // nw
    n_ch = per_w // ch
    mesh = plsc.VectorSubcoreMesh(core_axis_name="c", subcore_axis_name="s")

    @functools.partial(
        pl.kernel, mesh=mesh,
        out_type=jax.ShapeDtypeStruct((b_tot, d_feat), _F32),
        scratch_types=[pltpu.VMEM((ch,), jnp.int32),
                       pltpu.VMEM((ch, d_feat), _F32),
                       pltpu.SemaphoreType.DMA],
    )
    def gk(table_hbm, idx_hbm, out_hbm, idx_v, rows_v, sem):
        wid = lax.axis_index("s") * 2 + lax.axis_index("c")
        base = wid * per_w

        def body(c, carry):
            off = base + c * ch
            pltpu.sync_copy(idx_hbm.at[pl.ds(off, ch)], idx_v)
            pltpu.async_copy(table_hbm.at[idx_v], rows_v, sem).wait()
            pltpu.sync_copy(rows_v, out_hbm.at[pl.ds(off, ch)])
            return carry

        lax.fori_loop(0, n_ch, body, 0)

    return gk(table, idx_pad)


def _stage_first(hi, hj, w, b, d_in, d_out, be=2000):
    g = _NK // be
    d_hj = hj.shape[1]

    def body(hi_ref, hj_ref, w_ref, b_ref, y_ref):
        hi_v = hi_ref[...][:, :d_in]
        hj_v = hj_ref[...][:, :d_in]
        feat = jnp.concatenate([hi_v, hj_v - hi_v], axis=1)
        y_ref[...] = _lrelu(_dot(feat, w_ref[...]) + b_ref[...])

    return pl.pallas_call(
        body,
        grid=(g,),
        in_specs=[pl.BlockSpec((be, d_hj), lambda i: (i, 0)),
                  pl.BlockSpec((be, d_hj), lambda i: (i, 0)),
                  pl.BlockSpec((2 * d_in, d_out), lambda i: (0, 0)),
                  pl.BlockSpec((1, d_out), lambda i: (0, 0))],
        out_specs=pl.BlockSpec((be, d_out), lambda i: (i, 0)),
        out_shape=jax.ShapeDtypeStruct((_NK, d_out), _F32),
    )(hi, hj, w, b.reshape(1, -1))


def _stage_mid(xx, w, b, d_in, d_out, be=4000):
    g = _NK // be

    def body(x_ref, w_ref, b_ref, y_ref):
        y_ref[...] = _lrelu(_dot(x_ref[...], w_ref[...]) + b_ref[...])

    return pl.pallas_call(
        body,
        grid=(g,),
        in_specs=[pl.BlockSpec((be, d_in), lambda i: (i, 0)),
                  pl.BlockSpec((d_in, d_out), lambda i: (0, 0)),
                  pl.BlockSpec((1, d_out), lambda i: (0, 0))],
        out_specs=pl.BlockSpec((be, d_out), lambda i: (i, 0)),
        out_shape=jax.ShapeDtypeStruct((_NK, d_out), _F32),
    )(xx, w, b.reshape(1, -1))


def _stage_max(xx, d_feat, bn=1000):
    g = _N // bn
    x3 = xx.reshape(_N, _K, d_feat)

    def body(x_ref, y_ref):
        x = x_ref[...]
        m = x[:, 0, :]
        for k in range(1, _K):
            m = jnp.maximum(m, x[:, k, :])
        y_ref[...] = m

    return pl.pallas_call(
        body,
        grid=(g,),
        in_specs=[pl.BlockSpec((bn, _K, d_feat), lambda i: (i, 0, 0))],
        out_specs=pl.BlockSpec((bn, d_feat), lambda i: (i, 0)),
        out_shape=jax.ShapeDtypeStruct((_N, d_feat), _F32),
    )(x3)


def _node_head_body(g1_ref, g2_ref, box_ref, gowa, gowb, gob, gog, gobe,
                    ncw1, ncb1, ncw2, ncb2, ecwa, ecwb, ecb1,
                    logits_ref, a_ref, b_ref):
    h = _lrelu(_dot(g1_ref[...], gowa[...]) + _dot(g2_ref[...], gowb[...])
               + gob[...])
    mean = jnp.mean(h, axis=0, keepdims=True)
    var = jnp.mean((h - mean) ** 2, axis=0, keepdims=True)
    gg = (h - mean) / jnp.sqrt(var + _EPS) * gog[...] + gobe[...]
    nf = jnp.maximum(box_ref[...], gg)
    nl = _lrelu(_dot(nf, ncw1[...]) + ncb1[...])
    logits_ref[...] = jnp.maximum(_dot(nl, ncw2[...]) + ncb2[...], 0.0)
    a_ref[...] = _dot(nf, ecwa[...]) + ecb1[...]
    b_ref[...] = _dot(nf, ecwb[...])


def _node_head(g1, g2, box, p):
    r1 = lambda v: v.reshape(1, -1)
    return pl.pallas_call(
        _node_head_body,
        out_shape=[jax.ShapeDtypeStruct((_N, 32), _F32),
                   jax.ShapeDtypeStruct((_N, 128), _F32),
                   jax.ShapeDtypeStruct((_N, 128), _F32)],
    )(g1, g2, box, p['go_w'][:64], p['go_w'][64:], r1(p['go_b']),
      r1(p['go_g']), r1(p['go_be']), p['nc_w1'], r1(p['nc_b1']),
      p['nc_w2'], r1(p['nc_b2']), p['ec_w1'][:128], p['ec_w1'][128:256],
      r1(p['ec_b1']))


def _edge_head(ga, gb, ea, wc, w2, b2, be=4000):
    g = _E // be

    def body(a_ref, b_ref, ea_ref, wc_ref, w2_ref, b2_ref, o_ref):
        el = _lrelu(a_ref[...] + b_ref[...] + _dot(ea_ref[...], wc_ref[...]))
        o_ref[...] = jnp.maximum(_dot(el, w2_ref[...]) + b2_ref[...], 0.0)

    return pl.pallas_call(
        body,
        grid=(g,),
        in_specs=[pl.BlockSpec((be, 128), lambda i: (i, 0)),
                  pl.BlockSpec((be, 128), lambda i: (i, 0)),
                  pl.BlockSpec((be, 16), lambda i: (i, 0)),
                  pl.BlockSpec((16, 128), lambda i: (0, 0)),
                  pl.BlockSpec((128, 8), lambda i: (0, 0)),
                  pl.BlockSpec((1, 8), lambda i: (0, 0))],
        out_specs=pl.BlockSpec((be, 8), lambda i: (i, 0)),
        out_shape=jax.ShapeDtypeStruct((_E, 8), _F32),
    )(ga, gb, ea, wc, w2, b2.reshape(1, -1))


def _flat_idx(idx16):
    return jnp.pad(idx16[:, :_K].reshape(-1), (0, _NKPAD - _NK))


def kernel(x, nn_feature, edge_attr, params, edge_index, batch):
    p = params
    box, fusion = _prelude(_bn_host(x), _bn_host(nn_feature), p)
    rep = jnp.pad(jnp.arange(_NK, dtype=jnp.int32) // _K, (0, _NKPAD - _NK))

    idx1 = _flat_idx(_knn(fusion, 128))
    hi1 = _sc_gather(fusion, rep)
    hj1 = _sc_gather(fusion, idx1)
    z1 = _stage_first(hi1, hj1, p['c1_w1'], p['c1_b1'], 128, 64)
    z2 = _stage_mid(_bn_host(z1, p['c1_g1'], p['c1_be1']),
                    p['c1_w2'], p['c1_b2'], 64, 64)
    z3 = _stage_mid(_bn_host(z2, p['c1_g2'], p['c1_be2']),
                    p['c1_w3'], p['c1_b3'], 64, 64)
    m3, v3 = _bn_stats(z3)
    g1 = ((_stage_max(z3, 64) - m3) / jnp.sqrt(v3 + _EPS)
          * p['c1_g3'] + p['c1_be3'])

    idx2 = _flat_idx(_knn(g1, 64))
    g1p = jnp.pad(g1, ((0, 0), (0, 64)))
    hi2 = _sc_gather(g1p, rep)
    hj2 = _sc_gather(g1p, idx2)
    zz = _stage_first(hi2, hj2, p['c2_w1'], p['c2_b1'], 64, 128)
    mz, vz = _bn_stats(zz)
    g2 = ((_stage_max(zz, 128) - mz) / jnp.sqrt(vz + _EPS)
          * p['c2_g1'] + p['c2_be1'])

    node_logits, ea_tab, eb_tab = _node_head(g1, g2, box, p)
    rowp = jnp.pad(edge_index[0], (0, _EPAD - _E))
    colp = jnp.pad(edge_index[1], (0, _EPAD - _E))
    ga = _sc_gather(ea_tab, rowp)
    gb = _sc_gather(eb_tab, colp)
    edge_logits = _edge_head(ga, gb, edge_attr, p['ec_w1'][256:],
                             p['ec_w2'], p['ec_b2'])
    return (node_logits, edge_logits)
